# Initial kernel scaffold; baseline (speedup 1.0000x reference)
import jax
import jax.numpy as jnp
from jax.experimental import pallas as pl


def kernel(x, edge_index, W1, b1, W2, b2):
    raise NotImplementedError("write your pallas kernel here")


# speedup vs baseline: 12.4848x; 12.4848x over previous
import functools

import jax
import jax.numpy as jnp
from jax import lax
from jax.experimental import pallas as pl
from jax.experimental.pallas import tpu as pltpu
from jax.experimental.pallas import tpu_sc as plsc

N = 10000
E = 320000
NPAD = 10240
NC = 2
NS = 16
NW = NC * NS
EPW = E // NW
C = 80
NCHUNK = EPW // C
RPT = NPAD // NS
RB = 400

_mesh = plsc.VectorSubcoreMesh(core_axis_name="c", subcore_axis_name="s")


@functools.partial(
    pl.kernel,
    mesh=_mesh,
    out_type=jax.ShapeDtypeStruct((NC, NPAD), jnp.float32),
    scratch_types=[
        pltpu.VMEM((C,), jnp.int32),
        pltpu.VMEM((C,), jnp.float32),
        pltpu.VMEM((RPT,), jnp.float32),
        pltpu.VMEM_SHARED((NPAD,), jnp.float32),
    ],
)
def _sc_degree(dst_hbm, out_hbm, idx_v, ones_v, buf_v, deg_sh):
    c = lax.axis_index("c")
    s = lax.axis_index("s")
    wid = s * NC + c
    base = wid * EPW

    one16 = jnp.full((16,), 1.0, jnp.float32)
    zero16 = jnp.zeros((16,), jnp.float32)
    for j in range(C // 16):
        ones_v[pl.ds(j * 16, 16)] = one16

    def zb(i, _):
        buf_v[pl.ds(i * 16, 16)] = zero16
        return 0

    lax.fori_loop(0, RPT // 16, zb, 0)
    sl = pl.ds(s * RPT, RPT)
    pltpu.sync_copy(buf_v, deg_sh.at[sl])
    plsc.subcore_barrier()

    def body(i, _):
        pltpu.sync_copy(dst_hbm.at[pl.ds(base + i * C, C)], idx_v)
        pltpu.sync_copy(ones_v, deg_sh.at[idx_v], add=True)
        return 0

    lax.fori_loop(0, NCHUNK, body, 0)
    plsc.subcore_barrier()
    pltpu.sync_copy(deg_sh.at[sl], buf_v)
    pltpu.sync_copy(buf_v, out_hbm.at[c, sl])


def _make_sc_scatter(D):
    @functools.partial(
        pl.kernel,
        mesh=_mesh,
        out_type=jax.ShapeDtypeStruct((NC, NPAD, D), jnp.float32),
        scratch_types=[
            pltpu.VMEM((C,), jnp.int32),
            pltpu.VMEM((C,), jnp.int32),
            pltpu.VMEM((C, D), jnp.float32),
            pltpu.VMEM_SHARED((NPAD, D), jnp.float32),
            pltpu.SemaphoreType.DMA,
        ],
    )
    def scat(src_hbm, dst_hbm, z_hbm, out_hbm, src_v, dst_v, rows_v, acc_sh, sem):
        c = lax.axis_index("c")
        s = lax.axis_index("s")
        wid = s * NC + c
        base = wid * EPW

        zero16 = jnp.zeros((16,), jnp.float32)

        def zb(i, _):
            for j in range(D // 16):
                rows_v[i, pl.ds(j * 16, 16)] = zero16
            return 0

        lax.fori_loop(0, C, zb, 0)
        for j in range(RPT // C):
            pltpu.sync_copy(rows_v, acc_sh.at[pl.ds(s * RPT + j * C, C)])
        plsc.subcore_barrier()

        def body(i, _):
            off = base + i * C
            pltpu.sync_copy(src_hbm.at[pl.ds(off, C)], src_v)
            pltpu.sync_copy(dst_hbm.at[pl.ds(off, C)], dst_v)
            pltpu.async_copy(z_hbm.at[src_v], rows_v, sem).wait()
            pltpu.sync_copy(rows_v, acc_sh.at[dst_v], add=True)
            return 0

        lax.fori_loop(0, NCHUNK, body, 0)
        plsc.subcore_barrier()
        for j in range(RPT // C):
            sl = pl.ds(s * RPT + j * C, C)
            pltpu.sync_copy(acc_sh.at[sl], rows_v)
            pltpu.sync_copy(rows_v, out_hbm.at[c, sl])

    return scat


_sc_scatter_128 = _make_sc_scatter(128)


def _dense1_body(x_ref, w_ref, d0_ref, d1_ref, z_ref, dis_ref):
    deg = d0_ref[...] + d1_ref[...] + 1.0
    dis = lax.rsqrt(deg)
    dis_ref[...] = dis
    y = jnp.dot(x_ref[...], w_ref[...], preferred_element_type=jnp.float32)
    z_ref[...] = y * dis


def _dense1(x, W1, d0, d1):
    return pl.pallas_call(
        _dense1_body,
        grid=(N // RB,),
        in_specs=[
            pl.BlockSpec((RB, 128), lambda i: (i, 0)),
            pl.BlockSpec((128, 128), lambda i: (0, 0)),
            pl.BlockSpec((RB, 1), lambda i: (i, 0)),
            pl.BlockSpec((RB, 1), lambda i: (i, 0)),
        ],
        out_specs=[
            pl.BlockSpec((RB, 128), lambda i: (i, 0)),
            pl.BlockSpec((RB, 1), lambda i: (i, 0)),
        ],
        out_shape=[
            jax.ShapeDtypeStruct((N, 128), jnp.float32),
            jax.ShapeDtypeStruct((N, 1), jnp.float32),
        ],
    )(x, W1, d0, d1)


def _dense2_body(p0_ref, p1_ref, z_ref, dis_ref, b_ref, w_ref, z2_ref):
    dis = dis_ref[...]
    h = dis * (p0_ref[0] + p1_ref[0] + z_ref[...]) + b_ref[...]
    h = jnp.maximum(h, 0.0)
    y = jnp.dot(h, w_ref[...], preferred_element_type=jnp.float32)
    z2_ref[...] = jnp.concatenate(
        [y * dis, jnp.zeros((y.shape[0], 64), jnp.float32)], axis=1)


def _dense2(s1, z1, dis, b1, W2):
    return pl.pallas_call(
        _dense2_body,
        grid=(N // RB,),
        in_specs=[
            pl.BlockSpec((1, RB, 128), lambda i: (0, i, 0)),
            pl.BlockSpec((1, RB, 128), lambda i: (1, i, 0)),
            pl.BlockSpec((RB, 128), lambda i: (i, 0)),
            pl.BlockSpec((RB, 1), lambda i: (i, 0)),
            pl.BlockSpec((1, 128), lambda i: (0, 0)),
            pl.BlockSpec((128, 64), lambda i: (0, 0)),
        ],
        out_specs=pl.BlockSpec((RB, 128), lambda i: (i, 0)),
        out_shape=jax.ShapeDtypeStruct((N, 128), jnp.float32),
    )(s1, s1, z1, dis, b1, W2)


def _dense3_body(p0_ref, p1_ref, z_ref, dis_ref, b_ref, o_ref):
    t = (p0_ref[0] + p1_ref[0] + z_ref[...])[:, :64]
    h = dis_ref[...] * t + b_ref[...]
    o_ref[...] = jnp.maximum(h, 0.0)


def _dense3(s2, z2, dis, b2):
    return pl.pallas_call(
        _dense3_body,
        grid=(N // RB,),
        in_specs=[
            pl.BlockSpec((1, RB, 128), lambda i: (0, i, 0)),
            pl.BlockSpec((1, RB, 128), lambda i: (1, i, 0)),
            pl.BlockSpec((RB, 128), lambda i: (i, 0)),
            pl.BlockSpec((RB, 1), lambda i: (i, 0)),
            pl.BlockSpec((1, 64), lambda i: (0, 0)),
        ],
        out_specs=pl.BlockSpec((RB, 64), lambda i: (i, 0)),
        out_shape=jax.ShapeDtypeStruct((N, 64), jnp.float32),
    )(s2, s2, z2, dis, b2)


def _p0(a):
    return a[:, :N]


def kernel(x, edge_index, W1, b1, W2, b2):
    src = edge_index[0].astype(jnp.int32)
    dst = edge_index[1].astype(jnp.int32)

    degp = _sc_degree(dst)
    d0 = degp[0, :N].reshape(N, 1)
    d1 = degp[1, :N].reshape(N, 1)
    z1, dis = _dense1(x, W1, d0, d1)
    s1 = _sc_scatter_128(src, dst, z1)
    z2 = _dense2(s1[:, :N], z1, dis, b1.reshape(1, 128), W2)
    s2 = _sc_scatter_128(src, dst, z2)
    return _dense3(s2[:, :N], z2, dis, b2.reshape(1, 64))


# speedup vs baseline: 18.8348x; 1.5086x over previous
import functools

import jax
import jax.numpy as jnp
from jax import lax
from jax.experimental import pallas as pl
from jax.experimental.pallas import tpu as pltpu
from jax.experimental.pallas import tpu_sc as plsc

N = 10000
E = 320000
NPAD = 10240
NC = 2
NS = 16
NW = NC * NS
EPW = E // NW
C = 80
NCHUNK = EPW // C
RPT = NPAD // NS
RB = 400

_mesh = plsc.VectorSubcoreMesh(core_axis_name="c", subcore_axis_name="s")


@functools.partial(
    pl.kernel,
    mesh=_mesh,
    out_type=jax.ShapeDtypeStruct((NC, NPAD), jnp.float32),
    scratch_types=[
        pltpu.VMEM((C,), jnp.int32),
        pltpu.VMEM((C,), jnp.float32),
        pltpu.VMEM((RPT,), jnp.float32),
        pltpu.VMEM_SHARED((NPAD,), jnp.float32),
    ],
)
def _sc_degree(dst_hbm, out_hbm, idx_v, ones_v, buf_v, deg_sh):
    c = lax.axis_index("c")
    s = lax.axis_index("s")
    wid = s * NC + c
    base = wid * EPW

    one16 = jnp.full((16,), 1.0, jnp.float32)
    zero16 = jnp.zeros((16,), jnp.float32)
    for j in range(C // 16):
        ones_v[pl.ds(j * 16, 16)] = one16

    def zb(i, _):
        buf_v[pl.ds(i * 16, 16)] = zero16
        return 0

    lax.fori_loop(0, RPT // 16, zb, 0)
    sl = pl.ds(s * RPT, RPT)
    pltpu.sync_copy(buf_v, deg_sh.at[sl])
    plsc.subcore_barrier()

    def body(i, _):
        pltpu.sync_copy(dst_hbm.at[pl.ds(base + i * C, C)], idx_v)
        pltpu.sync_copy(ones_v, deg_sh.at[idx_v], add=True)
        return 0

    lax.fori_loop(0, NCHUNK, body, 0)
    plsc.subcore_barrier()
    pltpu.sync_copy(deg_sh.at[sl], out_hbm.at[c, sl])


def _make_sc_scatter(D):
    @functools.partial(
        pl.kernel,
        mesh=_mesh,
        out_type=jax.ShapeDtypeStruct((NC, NPAD, D), jnp.float32),
        scratch_types=[
            pltpu.VMEM((C,), jnp.int32),
            pltpu.VMEM((C,), jnp.int32),
            pltpu.VMEM((C,), jnp.int32),
            pltpu.VMEM((C,), jnp.int32),
            pltpu.VMEM((C, D), jnp.float32),
            pltpu.VMEM((C, D), jnp.float32),
            pltpu.VMEM_SHARED((NPAD, D), jnp.float32),
            pltpu.SemaphoreType.DMA,
            pltpu.SemaphoreType.DMA,
        ],
    )
    def scat(src_hbm, dst_hbm, z_hbm, out_hbm,
             src0, dst0, src1, dst1, rows0, rows1, acc_sh, sem0, sem1):
        c = lax.axis_index("c")
        s = lax.axis_index("s")
        base = (s * NC + c) * EPW

        zero16 = jnp.zeros((16,), jnp.float32)

        def zb(i, _):
            for j in range(D // 16):
                rows0[i, pl.ds(j * 16, 16)] = zero16
            return 0

        lax.fori_loop(0, C, zb, 0)
        for j in range(RPT // C):
            pltpu.sync_copy(rows0, acc_sh.at[pl.ds(s * RPT + j * C, C)])
        plsc.subcore_barrier()

        b0 = (src0, dst0, rows0, sem0)
        b1 = (src1, dst1, rows1, sem1)

        def start(i, b):
            sv, dv, rv, sem = b
            off = base + i * C
            pltpu.sync_copy(src_hbm.at[pl.ds(off, C)], sv)
            pltpu.sync_copy(dst_hbm.at[pl.ds(off, C)], dv)
            pltpu.async_copy(z_hbm.at[sv], rv, sem)

        def finish(b):
            sv, dv, rv, sem = b
            pltpu.make_async_copy(z_hbm.at[sv], rv, sem).wait()
            pltpu.sync_copy(rv, acc_sh.at[dv], add=True)

        start(0, b0)

        def body(j, _):
            i = 2 * j + 1
            start(i, b1)
            finish(b0)
            start(i + 1, b0)
            finish(b1)
            return 0

        lax.fori_loop(0, (NCHUNK - 1) // 2, body, 0)
        finish(b0)
        plsc.subcore_barrier()
        sl = pl.ds(s * RPT, RPT)
        pltpu.sync_copy(acc_sh.at[sl], out_hbm.at[c, sl])

    return scat


_sc_scatter_128 = _make_sc_scatter(128)


def _dense1_body(x_ref, w_ref, d0_ref, d1_ref, z_ref, dis_ref):
    deg = d0_ref[...] + d1_ref[...] + 1.0
    dis = lax.rsqrt(deg)
    dis_ref[...] = dis
    y = jnp.dot(x_ref[...], w_ref[...], preferred_element_type=jnp.float32)
    z_ref[...] = y * dis


def _dense1(x, W1, d0, d1):
    return pl.pallas_call(
        _dense1_body,
        grid=(N // RB,),
        in_specs=[
            pl.BlockSpec((RB, 128), lambda i: (i, 0)),
            pl.BlockSpec((128, 128), lambda i: (0, 0)),
            pl.BlockSpec((RB, 1), lambda i: (i, 0)),
            pl.BlockSpec((RB, 1), lambda i: (i, 0)),
        ],
        out_specs=[
            pl.BlockSpec((RB, 128), lambda i: (i, 0)),
            pl.BlockSpec((RB, 1), lambda i: (i, 0)),
        ],
        out_shape=[
            jax.ShapeDtypeStruct((N, 128), jnp.float32),
            jax.ShapeDtypeStruct((N, 1), jnp.float32),
        ],
    )(x, W1, d0, d1)


def _dense2_body(p0_ref, p1_ref, z_ref, dis_ref, b_ref, w_ref, z2_ref):
    dis = dis_ref[...]
    h = dis * (p0_ref[0] + p1_ref[0] + z_ref[...]) + b_ref[...]
    h = jnp.maximum(h, 0.0)
    y = jnp.dot(h, w_ref[...], preferred_element_type=jnp.float32)
    z2_ref[...] = jnp.concatenate(
        [y * dis, jnp.zeros((y.shape[0], 64), jnp.float32)], axis=1)


def _dense2(s1, z1, dis, b1, W2):
    return pl.pallas_call(
        _dense2_body,
        grid=(N // RB,),
        in_specs=[
            pl.BlockSpec((1, RB, 128), lambda i: (0, i, 0)),
            pl.BlockSpec((1, RB, 128), lambda i: (1, i, 0)),
            pl.BlockSpec((RB, 128), lambda i: (i, 0)),
            pl.BlockSpec((RB, 1), lambda i: (i, 0)),
            pl.BlockSpec((1, 128), lambda i: (0, 0)),
            pl.BlockSpec((128, 64), lambda i: (0, 0)),
        ],
        out_specs=pl.BlockSpec((RB, 128), lambda i: (i, 0)),
        out_shape=jax.ShapeDtypeStruct((N, 128), jnp.float32),
    )(s1, s1, z1, dis, b1, W2)


def _dense3_body(p0_ref, p1_ref, z_ref, dis_ref, b_ref, o_ref):
    t = (p0_ref[0] + p1_ref[0] + z_ref[...])[:, :64]
    h = dis_ref[...] * t + b_ref[...]
    o_ref[...] = jnp.maximum(h, 0.0)


def _dense3(s2, z2, dis, b2):
    return pl.pallas_call(
        _dense3_body,
        grid=(N // RB,),
        in_specs=[
            pl.BlockSpec((1, RB, 128), lambda i: (0, i, 0)),
            pl.BlockSpec((1, RB, 128), lambda i: (1, i, 0)),
            pl.BlockSpec((RB, 128), lambda i: (i, 0)),
            pl.BlockSpec((RB, 1), lambda i: (i, 0)),
            pl.BlockSpec((1, 64), lambda i: (0, 0)),
        ],
        out_specs=pl.BlockSpec((RB, 64), lambda i: (i, 0)),
        out_shape=jax.ShapeDtypeStruct((N, 64), jnp.float32),
    )(s2, s2, z2, dis, b2)


def _p0(a):
    return a[:, :N]


def kernel(x, edge_index, W1, b1, W2, b2):
    src = edge_index[0].astype(jnp.int32)
    dst = edge_index[1].astype(jnp.int32)

    degp = _sc_degree(dst)
    d0 = degp[0, :N].reshape(N, 1)
    d1 = degp[1, :N].reshape(N, 1)
    z1, dis = _dense1(x, W1, d0, d1)
    s1 = _sc_scatter_128(src, dst, z1)
    z2 = _dense2(s1, z1, dis, b1.reshape(1, 128), W2)
    s2 = _sc_scatter_128(src, dst, z2)
    return _dense3(s2, z2, dis, b2.reshape(1, 64))


# speedup vs baseline: 28.7592x; 1.5269x over previous
import functools

import jax
import jax.numpy as jnp
from jax import lax
from jax.experimental import pallas as pl
from jax.experimental.pallas import tpu as pltpu
from jax.experimental.pallas import tpu_sc as plsc

N = 10000
E = 320000
NPAD = 10240
NC = 2
NS = 16
NW = NC * NS
EPW = E // NW
C = 80
NCHUNK = EPW // C
RPT = NPAD // NS
RB = 400

_mesh = plsc.VectorSubcoreMesh(core_axis_name="c", subcore_axis_name="s")


@functools.partial(
    pl.kernel,
    mesh=_mesh,
    out_type=jax.ShapeDtypeStruct((NC, NPAD), jnp.float32),
    scratch_types=[
        pltpu.VMEM((NCHUNK, 1, C), jnp.int32),
        pltpu.VMEM((C,), jnp.float32),
        pltpu.VMEM((RPT,), jnp.float32),
        pltpu.VMEM_SHARED((NPAD,), jnp.float32),
        pltpu.SemaphoreType.DMA,
    ],
)
def _sc_degree(dst2d_hbm, out_hbm, dst_all, ones_v, buf_v, deg_sh, sem):
    c = lax.axis_index("c")
    s = lax.axis_index("s")
    wid = s * NC + c

    one16 = jnp.full((16,), 1.0, jnp.float32)
    zero16 = jnp.zeros((16,), jnp.float32)
    for j in range(C // 16):
        ones_v[pl.ds(j * 16, 16)] = one16

    def zb(i, _):
        buf_v[pl.ds(i * 16, 16)] = zero16
        return 0

    lax.fori_loop(0, RPT // 16, zb, 0)
    sl = pl.ds(s * RPT, RPT)
    pltpu.sync_copy(buf_v, deg_sh.at[sl])
    pltpu.sync_copy(dst2d_hbm.at[pl.ds(wid * NCHUNK, NCHUNK)], dst_all)
    plsc.subcore_barrier()

    def body(i, _):
        pltpu.async_copy(ones_v, deg_sh.at[dst_all.at[i, 0]], sem, add=True)
        return 0

    lax.fori_loop(0, NCHUNK, body, 0)

    def drain(i, _):
        pltpu.make_async_copy(ones_v, deg_sh.at[dst_all.at[i, 0]], sem).wait()
        return 0

    lax.fori_loop(0, NCHUNK, drain, 0)
    plsc.subcore_barrier()
    pltpu.sync_copy(deg_sh.at[sl], out_hbm.at[c, sl])


def _make_sc_scatter(D):
    @functools.partial(
        pl.kernel,
        mesh=_mesh,
        out_type=jax.ShapeDtypeStruct((NC, NPAD, D), jnp.float32),
        scratch_types=[
            pltpu.VMEM((EPW,), jnp.int32),
            pltpu.VMEM((NCHUNK, 1, C), jnp.int32),
            pltpu.VMEM((C, D), jnp.float32),
            pltpu.VMEM((C, D), jnp.float32),
            pltpu.VMEM_SHARED((NPAD, D), jnp.float32),
            pltpu.SemaphoreType.DMA,
            pltpu.SemaphoreType.DMA,
        ],
    )
    def scat(src_hbm, dst2d_hbm, z_hbm, out_hbm,
             src_all, dst_all, rows0, rows1, acc_sh, sem0, sem1):
        c = lax.axis_index("c")
        s = lax.axis_index("s")
        wid = s * NC + c
        base = wid * EPW

        zero16 = jnp.zeros((16,), jnp.float32)

        def zb(i, _):
            for j in range(D // 16):
                rows0[i, pl.ds(j * 16, 16)] = zero16
            return 0

        lax.fori_loop(0, C, zb, 0)
        for j in range(RPT // C):
            pltpu.sync_copy(rows0, acc_sh.at[pl.ds(s * RPT + j * C, C)])
        pltpu.sync_copy(src_hbm.at[pl.ds(base, EPW)], src_all)
        pltpu.sync_copy(dst2d_hbm.at[pl.ds(wid * NCHUNK, NCHUNK)], dst_all)
        plsc.subcore_barrier()

        b0 = (rows0, sem0)
        b1 = (rows1, sem1)

        def start(i, b):
            rv, sem = b
            pltpu.async_copy(z_hbm.at[src_all.at[pl.ds(i * C, C)]], rv, sem)

        def finish(i, b):
            rv, sem = b
            pltpu.make_async_copy(
                z_hbm.at[src_all.at[pl.ds(i * C, C)]], rv, sem).wait()
            pltpu.sync_copy(rv, acc_sh.at[dst_all.at[i, 0]], add=True)

        start(0, b0)

        def body(j, _):
            i = 2 * j + 1
            start(i, b1)
            finish(i - 1, b0)
            start(i + 1, b0)
            finish(i, b1)
            return 0

        lax.fori_loop(0, (NCHUNK - 1) // 2, body, 0)
        finish(NCHUNK - 1, b0)
        plsc.subcore_barrier()
        sl = pl.ds(s * RPT, RPT)
        pltpu.sync_copy(acc_sh.at[sl], out_hbm.at[c, sl])

    return scat


_sc_scatter_128 = _make_sc_scatter(128)


def _dense1_body(x_ref, w_ref, d0_ref, d1_ref, z_ref, dis_ref):
    deg = d0_ref[...] + d1_ref[...] + 1.0
    dis = lax.rsqrt(deg)
    dis_ref[...] = dis
    y = jnp.dot(x_ref[...], w_ref[...], preferred_element_type=jnp.float32)
    z_ref[...] = y * dis


def _dense1(x, W1, d0, d1):
    return pl.pallas_call(
        _dense1_body,
        grid=(N // RB,),
        in_specs=[
            pl.BlockSpec((RB, 128), lambda i: (i, 0)),
            pl.BlockSpec((128, 128), lambda i: (0, 0)),
            pl.BlockSpec((RB, 1), lambda i: (i, 0)),
            pl.BlockSpec((RB, 1), lambda i: (i, 0)),
        ],
        out_specs=[
            pl.BlockSpec((RB, 128), lambda i: (i, 0)),
            pl.BlockSpec((RB, 1), lambda i: (i, 0)),
        ],
        out_shape=[
            jax.ShapeDtypeStruct((N, 128), jnp.float32),
            jax.ShapeDtypeStruct((N, 1), jnp.float32),
        ],
    )(x, W1, d0, d1)


def _dense2_body(p0_ref, p1_ref, z_ref, dis_ref, b_ref, w_ref, z2_ref):
    dis = dis_ref[...]
    h = dis * (p0_ref[0] + p1_ref[0] + z_ref[...]) + b_ref[...]
    h = jnp.maximum(h, 0.0)
    y = jnp.dot(h, w_ref[...], preferred_element_type=jnp.float32)
    z2_ref[...] = jnp.concatenate(
        [y * dis, jnp.zeros((y.shape[0], 64), jnp.float32)], axis=1)


def _dense2(s1, z1, dis, b1, W2):
    return pl.pallas_call(
        _dense2_body,
        grid=(N // RB,),
        in_specs=[
            pl.BlockSpec((1, RB, 128), lambda i: (0, i, 0)),
            pl.BlockSpec((1, RB, 128), lambda i: (1, i, 0)),
            pl.BlockSpec((RB, 128), lambda i: (i, 0)),
            pl.BlockSpec((RB, 1), lambda i: (i, 0)),
            pl.BlockSpec((1, 128), lambda i: (0, 0)),
            pl.BlockSpec((128, 64), lambda i: (0, 0)),
        ],
        out_specs=pl.BlockSpec((RB, 128), lambda i: (i, 0)),
        out_shape=jax.ShapeDtypeStruct((N, 128), jnp.float32),
    )(s1, s1, z1, dis, b1, W2)


def _dense3_body(p0_ref, p1_ref, z_ref, dis_ref, b_ref, o_ref):
    t = (p0_ref[0] + p1_ref[0] + z_ref[...])[:, :64]
    h = dis_ref[...] * t + b_ref[...]
    o_ref[...] = jnp.maximum(h, 0.0)


def _dense3(s2, z2, dis, b2):
    return pl.pallas_call(
        _dense3_body,
        grid=(N // RB,),
        in_specs=[
            pl.BlockSpec((1, RB, 128), lambda i: (0, i, 0)),
            pl.BlockSpec((1, RB, 128), lambda i: (1, i, 0)),
            pl.BlockSpec((RB, 128), lambda i: (i, 0)),
            pl.BlockSpec((RB, 1), lambda i: (i, 0)),
            pl.BlockSpec((1, 64), lambda i: (0, 0)),
        ],
        out_specs=pl.BlockSpec((RB, 64), lambda i: (i, 0)),
        out_shape=jax.ShapeDtypeStruct((N, 64), jnp.float32),
    )(s2, s2, z2, dis, b2)


def _p0(a):
    return a[:, :N]


def kernel(x, edge_index, W1, b1, W2, b2):
    src = edge_index[0].astype(jnp.int32)
    dst2d = edge_index[1].astype(jnp.int32).reshape(NW * NCHUNK, 1, C)

    degp = _sc_degree(dst2d)
    d0 = degp[0, :N].reshape(N, 1)
    d1 = degp[1, :N].reshape(N, 1)
    z1, dis = _dense1(x, W1, d0, d1)
    s1 = _sc_scatter_128(src, dst2d, z1)
    z2 = _dense2(s1, z1, dis, b1.reshape(1, 128), W2)
    s2 = _sc_scatter_128(src, dst2d, z2)
    return _dense3(s2, z2, dis, b2.reshape(1, 64))
